# baseline (speedup 1.0000x reference)
import functools

import jax
import jax.numpy as jnp
from jax import lax
from jax.experimental import pallas as pl
from jax.experimental.pallas import tpu as pltpu
from jax.experimental.pallas import tpu_sc as plsc

_NUM_PROTEINS = 500000
_DC_R = 2.0
_DC_T = 1.0

_E = 8192
_D = 64
_W = 128
_NW = 32
_EPW = _E // _NW
_CHUNK = 128
_NCHUNK = _EPW // _CHUNK
_L = 16


def _sc_body(hr_hbm, idx0_hbm, idx1_hbm, out_hbm,
             idx0_v, idx1_v, row0_v, row1_v,
             r0a, r1a, r0b, r1b, out_v, sem0a, sem1a, sem0b, sem1b):
    wid = lax.axis_index("s") * 2 + lax.axis_index("c")
    base = wid * _EPW

    pltpu.sync_copy(idx0_hbm.at[pl.ds(base, _EPW)], idx0_v)
    pltpu.sync_copy(idx1_hbm.at[pl.ds(base, _EPW)], idx1_v)

    for j in range(_EPW // _L):
        sl = pl.ds(j * _L, _L)
        v1 = idx1_v[sl] + _NUM_PROTEINS
        idx1_v[sl] = v1
        row0_v[sl] = lax.shift_right_logical(idx0_v[sl], 1)
        row1_v[sl] = lax.shift_right_logical(v1, 1)

    iota = lax.iota(jnp.int32, _L)

    def issue(c, r0, r1, s0, s1):
        sl = pl.ds(c * _CHUNK, _CHUNK)
        pltpu.async_copy(hr_hbm.at[row0_v.at[sl]], r0, s0)
        pltpu.async_copy(hr_hbm.at[row1_v.at[sl]], r1, s1)

    def drain(r0, r1, s0, s1):
        pltpu.make_async_copy(hr_hbm.at[pl.ds(0, _CHUNK)], r0, s0).wait()
        pltpu.make_async_copy(hr_hbm.at[pl.ds(0, _CHUNK)], r1, s1).wait()

    def compute(c, r0, r1):
        for g in range(_CHUNK // _L):
            row_ids = iota + g * _L
            esl = pl.ds(c * _CHUNK + g * _L, _L)
            col0 = (idx0_v[esl] & 1) * _D
            col1 = (idx1_v[esl] & 1) * _D

            def dim_step(d, acc):
                a = plsc.load_gather(r0, [row_ids, col0 + d])
                b = plsc.load_gather(r1, [row_ids, col1 + d])
                diff = a - b
                return acc + diff * diff

            sqdist = lax.fori_loop(0, _D, dim_step,
                                   jnp.zeros((_L,), jnp.float32))
            probs = 1.0 / (jnp.exp((sqdist - _DC_R) / _DC_T) + 1.0)
            out_v[esl] = probs

    bufs = [(r0a, r1a, sem0a, sem1a), (r0b, r1b, sem0b, sem1b)]
    issue(0, *bufs[0])
    for c in range(_NCHUNK):
        if c + 1 < _NCHUNK:
            issue(c + 1, *bufs[(c + 1) % 2])
        r0, r1, s0, s1 = bufs[c % 2]
        drain(r0, r1, s0, s1)
        compute(c, r0, r1)

    pltpu.sync_copy(out_v, out_hbm.at[pl.ds(base, _EPW)])


@jax.jit
def _run(hr, idx0, idx1):
    mesh = plsc.VectorSubcoreMesh(core_axis_name="c", subcore_axis_name="s")
    f = functools.partial(
        pl.kernel, mesh=mesh,
        out_type=jax.ShapeDtypeStruct((_E,), jnp.float32),
        scratch_types=[
            pltpu.VMEM((_EPW,), jnp.int32),
            pltpu.VMEM((_EPW,), jnp.int32),
            pltpu.VMEM((_EPW,), jnp.int32),
            pltpu.VMEM((_EPW,), jnp.int32),
            pltpu.VMEM((_CHUNK, _W), jnp.float32),
            pltpu.VMEM((_CHUNK, _W), jnp.float32),
            pltpu.VMEM((_CHUNK, _W), jnp.float32),
            pltpu.VMEM((_CHUNK, _W), jnp.float32),
            pltpu.VMEM((_EPW,), jnp.float32),
            pltpu.SemaphoreType.DMA,
            pltpu.SemaphoreType.DMA,
            pltpu.SemaphoreType.DMA,
            pltpu.SemaphoreType.DMA,
        ],
        compiler_params=pltpu.CompilerParams(
            needs_layout_passes=False, use_tc_tiling_on_sc=True),
    )(_sc_body)
    return f(hr, idx0, idx1)


def kernel(h, idx):
    idx32 = idx.astype(jnp.int32)
    hr = h.reshape(_NUM_PROTEINS, _W)
    pa = _run(hr, idx32[:_E, 0], idx32[:_E, 1])
    pb = _run(hr, idx32[_E:, 0], idx32[_E:, 1])
    return jnp.concatenate([pa, pb])


# speedup vs baseline: 1.0174x; 1.0174x over previous
import functools

import jax
import jax.numpy as jnp
from jax import lax
from jax.experimental import pallas as pl
from jax.experimental.pallas import tpu as pltpu
from jax.experimental.pallas import tpu_sc as plsc

_NUM_PROTEINS = 500000
_DC_R = 2.0
_DC_T = 1.0

_E = 16384
_D = 64
_W = 128
_NW = 32
_EPW = _E // _NW
_CHUNK = 128
_NCHUNK = _EPW // _CHUNK
_L = 16


def _sc_body(hr_hbm, idx0_hbm, idx1_hbm, out_hbm,
             idx0_v, idx1_v, row0_v, row1_v,
             r0a, r1a, r0b, r1b, out_v, sem0a, sem1a, sem0b, sem1b):
    wid = lax.axis_index("s") * 2 + lax.axis_index("c")
    base = wid * _EPW

    pltpu.sync_copy(idx0_hbm.at[pl.ds(base, _EPW)], idx0_v)
    pltpu.sync_copy(idx1_hbm.at[pl.ds(base, _EPW)], idx1_v)

    for j in range(_EPW // _L):
        sl = pl.ds(j * _L, _L)
        v1 = idx1_v[sl] + _NUM_PROTEINS
        idx1_v[sl] = v1
        row0_v[sl] = lax.shift_right_logical(idx0_v[sl], 1)
        row1_v[sl] = lax.shift_right_logical(v1, 1)

    iota = lax.iota(jnp.int32, _L)

    def issue(c, r0, r1, s0, s1):
        sl = pl.ds(c * _CHUNK, _CHUNK)
        pltpu.async_copy(hr_hbm.at[row0_v.at[sl]], r0, s0)
        pltpu.async_copy(hr_hbm.at[row1_v.at[sl]], r1, s1)

    def drain(r0, r1, s0, s1):
        pltpu.make_async_copy(hr_hbm.at[pl.ds(0, _CHUNK)], r0, s0).wait()
        pltpu.make_async_copy(hr_hbm.at[pl.ds(0, _CHUNK)], r1, s1).wait()

    def compute(c, r0, r1):
        for g in range(_CHUNK // _L):
            row_ids = iota + g * _L
            esl = pl.ds(c * _CHUNK + g * _L, _L)
            col0 = (idx0_v[esl] & 1) * _D
            col1 = (idx1_v[esl] & 1) * _D

            def dim_step(d, acc):
                a = plsc.load_gather(r0, [row_ids, col0 + d])
                b = plsc.load_gather(r1, [row_ids, col1 + d])
                diff = a - b
                return acc + diff * diff

            sqdist = lax.fori_loop(0, _D, dim_step,
                                   jnp.zeros((_L,), jnp.float32))
            probs = 1.0 / (jnp.exp((sqdist - _DC_R) / _DC_T) + 1.0)
            out_v[esl] = probs

    bufs = [(r0a, r1a, sem0a, sem1a), (r0b, r1b, sem0b, sem1b)]
    issue(0, *bufs[0])
    for c in range(_NCHUNK):
        if c + 1 < _NCHUNK:
            issue(c + 1, *bufs[(c + 1) % 2])
        r0, r1, s0, s1 = bufs[c % 2]
        drain(r0, r1, s0, s1)
        compute(c, r0, r1)

    pltpu.sync_copy(out_v, out_hbm.at[pl.ds(base, _EPW)])


@jax.jit
def _run(hr, idx0, idx1):
    mesh = plsc.VectorSubcoreMesh(core_axis_name="c", subcore_axis_name="s")
    f = functools.partial(
        pl.kernel, mesh=mesh,
        out_type=jax.ShapeDtypeStruct((_E,), jnp.float32),
        scratch_types=[
            pltpu.VMEM((_EPW,), jnp.int32),
            pltpu.VMEM((_EPW,), jnp.int32),
            pltpu.VMEM((_EPW,), jnp.int32),
            pltpu.VMEM((_EPW,), jnp.int32),
            pltpu.VMEM((_CHUNK, _W), jnp.float32),
            pltpu.VMEM((_CHUNK, _W), jnp.float32),
            pltpu.VMEM((_CHUNK, _W), jnp.float32),
            pltpu.VMEM((_CHUNK, _W), jnp.float32),
            pltpu.VMEM((_EPW,), jnp.float32),
            pltpu.SemaphoreType.DMA,
            pltpu.SemaphoreType.DMA,
            pltpu.SemaphoreType.DMA,
            pltpu.SemaphoreType.DMA,
        ],
        compiler_params=pltpu.CompilerParams(
            needs_layout_passes=False, use_tc_tiling_on_sc=True),
    )(_sc_body)
    return f(hr, idx0, idx1)


def kernel(h, idx):
    idx32 = idx.astype(jnp.int32)
    hr = h.reshape(_NUM_PROTEINS, _W)
    return _run(hr, idx32[:, 0], idx32[:, 1])
